# baseline (speedup 1.0000x reference)
import functools

import jax
import jax.numpy as jnp
from jax import lax
from jax.experimental import pallas as pl
from jax.experimental.pallas import tpu as pltpu
from jax.experimental.pallas import tpu_sc as plsc

N = 10000
E = 320000
D = 128
NC = 2
NS = 16
NW = NC * NS
EPW = E // NW
K = 88
NCHUNK = 114
EPW_PAD = NCHUNK * K
ACC_ROWS = 10016
NB = 4
NI = 8
MAIN = 112
RPT = 624
REM = N - RPT * NS


def _sc_partial(data, se, de, zeros):
    mesh = plsc.VectorSubcoreMesh(
        core_axis_name="c", subcore_axis_name="s", num_cores=NC
    )

    @functools.partial(
        pl.kernel,
        out_type=jax.ShapeDtypeStruct((NC, N, D), jnp.float32),
        mesh=mesh,
        scratch_types=[pltpu.VMEM_SHARED((ACC_ROWS, D), jnp.float32)]
        + [pltpu.VMEM((K,), jnp.int32) for _ in range(2 * NI)]
        + [pltpu.VMEM((K, D), jnp.float32) for _ in range(NB)]
        + [pltpu.SemaphoreType.DMA for _ in range(NI + 2 * NB)],
    )
    def k(data_hbm, se_hbm, de_hbm, zero_hbm, out_hbm, acc, *scr):
        srcb = scr[0:NI]
        dstb = scr[NI:2 * NI]
        rows = scr[2 * NI:2 * NI + NB]
        isem = scr[2 * NI + NB:2 * NI + NB + NI]
        gsem = scr[2 * NI + NB + NI:2 * NI + NB + NI + NB]
        ssem = scr[2 * NI + NB + NI + NB:]
        c = lax.axis_index("c")
        s = lax.axis_index("s")
        wid = s * NC + c

        pltpu.sync_copy(
            zero_hbm.at[pl.ds(s * RPT, RPT)], acc.at[pl.ds(s * RPT, RPT)]
        )

        @pl.when(s == 0)
        def _zero_rem():
            pltpu.sync_copy(
                zero_hbm.at[pl.ds(RPT * NS, REM)], acc.at[pl.ds(RPT * NS, REM)]
            )

        plsc.subcore_barrier()

        base0 = wid * EPW_PAD

        def start_idx(g, bi):
            pltpu.async_copy(se_hbm.at[pl.ds(base0 + g * K, K)], srcb[bi], isem[bi])
            pltpu.async_copy(de_hbm.at[pl.ds(base0 + g * K, K)], dstb[bi], isem[bi])

        def wait_idx(g, bi):
            pltpu.make_async_copy(
                se_hbm.at[pl.ds(base0 + g * K, K)], srcb[bi], isem[bi]
            ).wait()
            pltpu.make_async_copy(
                de_hbm.at[pl.ds(base0 + g * K, K)], dstb[bi], isem[bi]
            ).wait()

        def start_gather(b, bi):
            pltpu.async_copy(data_hbm.at[srcb[bi]], rows[b], gsem[b])

        def wait_gather(b, bi):
            pltpu.make_async_copy(data_hbm.at[srcb[bi]], rows[b], gsem[b]).wait()

        def start_scatter(b, bi):
            pltpu.async_copy(rows[b], acc.at[dstb[bi]], ssem[b], add=True)

        def wait_scatter(b, bi):
            pltpu.make_async_copy(rows[b], acc.at[dstb[bi]], ssem[b]).wait()

        for g in range(NB):
            start_idx(g, g % NI)
        for g in range(NB - 1):
            wait_idx(g, g % NI)
            start_gather(g % NB, g % NI)

        for g in range(NI):
            wait_gather(g % NB, g % NI)
            if g > 0:
                wait_scatter((g - 1) % NB, (g - 1) % NI)
            wait_idx(g + 3, (g + 3) % NI)
            start_gather((g + 3) % NB, (g + 3) % NI)
            start_scatter(g % NB, g % NI)
            start_idx(g + NB, (g + NB) % NI)

        @pl.loop(NI, MAIN, step=NI)
        def _grp(g0):
            for j in range(NI):
                g = g0 + j
                wait_gather(j % NB, j)
                wait_scatter((j - 1) % NB, (j - 1) % NI)
                wait_idx(g + 3, (j + 3) % NI)
                start_gather((j + 3) % NB, (j + 3) % NI)
                start_scatter(j % NB, j)
                start_idx(g + NB, (j + NB) % NI)

        for g in range(MAIN, NCHUNK):
            b = g % NB
            bi = g % NI
            wait_gather(b, bi)
            wait_scatter((g - 1) % NB, (g - 1) % NI)
            if g + 3 < NCHUNK:
                wait_idx(g + 3, (g + 3) % NI)
                start_gather((g + 3) % NB, (g + 3) % NI)
            start_scatter(b, bi)
            if g + NB < NCHUNK:
                start_idx(g + NB, (g + NB) % NI)
        wait_scatter((NCHUNK - 1) % NB, (NCHUNK - 1) % NI)

        plsc.subcore_barrier()
        pltpu.sync_copy(
            acc.at[pl.ds(s * RPT, RPT)], out_hbm.at[c, pl.ds(s * RPT, RPT)]
        )

        @pl.when(s == 0)
        def _flush_rem():
            pltpu.sync_copy(
                acc.at[pl.ds(RPT * NS, REM)], out_hbm.at[c, pl.ds(RPT * NS, REM)]
            )

    return k(data, se, de, zeros)


def _combine(partial):
    def body(p_ref, o_ref):
        o_ref[...] = p_ref[0] + p_ref[1]

    return pl.pallas_call(
        body,
        out_shape=jax.ShapeDtypeStruct((N, D), jnp.float32),
        grid=(10,),
        in_specs=[pl.BlockSpec((2, 1000, D), lambda i: (0, i, 0))],
        out_specs=pl.BlockSpec((1000, D), lambda i: (i, 0)),
    )(partial)


@jax.jit
def kernel(data, edge_index):
    pad = EPW_PAD - EPW
    src = edge_index[0].reshape(NW, EPW)
    dst = edge_index[1].reshape(NW, EPW)
    se = jnp.pad(src, ((0, 0), (0, pad))).reshape(NW * EPW_PAD)
    dump = (N + jnp.arange(NW, dtype=jnp.int32) // NC)[:, None]
    de = jnp.concatenate(
        [dst, jnp.broadcast_to(dump, (NW, pad))], axis=1
    ).reshape(NW * EPW_PAD)
    zeros = jnp.zeros((N, D), jnp.float32)
    partial = _sc_partial(data, se, de, zeros)
    return _combine(partial)


# speedup vs baseline: 1.2926x; 1.2926x over previous
import functools

import jax
import jax.numpy as jnp
from jax import lax
from jax.experimental import pallas as pl
from jax.experimental.pallas import tpu as pltpu
from jax.experimental.pallas import tpu_sc as plsc

N = 10000
E = 320000
D = 128
NC = 2
NS = 16
NW = NC * NS
EPW = E // NW
K = 80
NCHUNK = EPW // K
NB = 4
NI = 8
MAIN = 120
RPT = 624
REM = N - RPT * NS


def _sc_partial(data, se, de, zeros):
    mesh = plsc.VectorSubcoreMesh(
        core_axis_name="c", subcore_axis_name="s", num_cores=NC
    )

    @functools.partial(
        pl.kernel,
        out_type=jax.ShapeDtypeStruct((NC, N, D), jnp.float32),
        mesh=mesh,
        scratch_types=[pltpu.VMEM_SHARED((N, D), jnp.float32)]
        + [pltpu.VMEM((K,), jnp.int32) for _ in range(2 * NI)]
        + [pltpu.VMEM((K, D), jnp.float32) for _ in range(NB)]
        + [pltpu.SemaphoreType.DMA for _ in range(NI + 2 * NB)],
    )
    def k(data_hbm, se_hbm, de_hbm, zero_hbm, out_hbm, acc, *scr):
        srcb = scr[0:NI]
        dstb = scr[NI:2 * NI]
        rows = scr[2 * NI:2 * NI + NB]
        isem = scr[2 * NI + NB:2 * NI + NB + NI]
        gsem = scr[2 * NI + NB + NI:2 * NI + NB + NI + NB]
        ssem = scr[2 * NI + NB + NI + NB:]
        c = lax.axis_index("c")
        s = lax.axis_index("s")
        wid = s * NC + c

        pltpu.sync_copy(
            zero_hbm.at[pl.ds(s * RPT, RPT)], acc.at[pl.ds(s * RPT, RPT)]
        )

        @pl.when(s == 0)
        def _zero_rem():
            pltpu.sync_copy(
                zero_hbm.at[pl.ds(RPT * NS, REM)], acc.at[pl.ds(RPT * NS, REM)]
            )

        plsc.subcore_barrier()

        base0 = wid * EPW

        def start_idx(g, bi):
            pltpu.async_copy(se_hbm.at[pl.ds(base0 + g * K, K)], srcb[bi], isem[bi])
            pltpu.async_copy(de_hbm.at[pl.ds(base0 + g * K, K)], dstb[bi], isem[bi])

        def wait_idx(g, bi):
            pltpu.make_async_copy(
                se_hbm.at[pl.ds(base0 + g * K, K)], srcb[bi], isem[bi]
            ).wait()
            pltpu.make_async_copy(
                de_hbm.at[pl.ds(base0 + g * K, K)], dstb[bi], isem[bi]
            ).wait()

        def start_gather(b, bi):
            pltpu.async_copy(data_hbm.at[srcb[bi]], rows[b], gsem[b])

        def wait_gather(b, bi):
            pltpu.make_async_copy(data_hbm.at[srcb[bi]], rows[b], gsem[b]).wait()

        def start_scatter(b, bi):
            pltpu.async_copy(rows[b], acc.at[dstb[bi]], ssem[b], add=True)

        def wait_scatter(b, bi):
            pltpu.make_async_copy(rows[b], acc.at[dstb[bi]], ssem[b]).wait()

        for g in range(NB):
            start_idx(g, g % NI)
        for g in range(NB - 1):
            wait_idx(g, g % NI)
            start_gather(g % NB, g % NI)

        for g in range(NI):
            wait_gather(g % NB, g % NI)
            if g > 0:
                wait_scatter((g - 1) % NB, (g - 1) % NI)
            wait_idx(g + 3, (g + 3) % NI)
            start_gather((g + 3) % NB, (g + 3) % NI)
            start_scatter(g % NB, g % NI)
            start_idx(g + NB, (g + NB) % NI)

        @pl.loop(NI, MAIN, step=NI)
        def _grp(g0):
            for j in range(NI):
                g = g0 + j
                wait_gather(j % NB, j)
                wait_scatter((j - 1) % NB, (j - 1) % NI)
                wait_idx(g + 3, (j + 3) % NI)
                start_gather((j + 3) % NB, (j + 3) % NI)
                start_scatter(j % NB, j)
                start_idx(g + NB, (j + NB) % NI)

        for g in range(MAIN, NCHUNK):
            b = g % NB
            bi = g % NI
            wait_gather(b, bi)
            wait_scatter((g - 1) % NB, (g - 1) % NI)
            if g + 3 < NCHUNK:
                wait_idx(g + 3, (g + 3) % NI)
                start_gather((g + 3) % NB, (g + 3) % NI)
            start_scatter(b, bi)
            if g + NB < NCHUNK:
                start_idx(g + NB, (g + NB) % NI)
        wait_scatter((NCHUNK - 1) % NB, (NCHUNK - 1) % NI)

        plsc.subcore_barrier()
        pltpu.sync_copy(
            acc.at[pl.ds(s * RPT, RPT)], out_hbm.at[c, pl.ds(s * RPT, RPT)]
        )

        @pl.when(s == 0)
        def _flush_rem():
            pltpu.sync_copy(
                acc.at[pl.ds(RPT * NS, REM)], out_hbm.at[c, pl.ds(RPT * NS, REM)]
            )

    return k(data, se, de, zeros)


def _combine(partial):
    def body(p_ref, o_ref):
        o_ref[...] = p_ref[0] + p_ref[1]

    return pl.pallas_call(
        body,
        out_shape=jax.ShapeDtypeStruct((N, D), jnp.float32),
        grid=(10,),
        in_specs=[pl.BlockSpec((2, 1000, D), lambda i: (0, i, 0))],
        out_specs=pl.BlockSpec((1000, D), lambda i: (i, 0)),
    )(partial)


@jax.jit
def kernel(data, edge_index):
    se = edge_index[0]
    de = edge_index[1]
    zeros = jnp.zeros((N, D), jnp.float32)
    partial = _sc_partial(data, se, de, zeros)
    return _combine(partial)


# speedup vs baseline: 1.3136x; 1.0162x over previous
import functools

import jax
import jax.numpy as jnp
from jax import lax
from jax.experimental import pallas as pl
from jax.experimental.pallas import tpu as pltpu
from jax.experimental.pallas import tpu_sc as plsc

N = 10000
E = 320000
D = 128
NC = 2
NS = 16
NW = NC * NS
EPW = E // NW
K = 80
NCHUNK = EPW // K
NB = 4
NI = 8
MAIN = 120
RPT = 624
REM = N - RPT * NS


def _sc_partial(data, se, de, zeros):
    mesh = plsc.VectorSubcoreMesh(
        core_axis_name="c", subcore_axis_name="s", num_cores=NC
    )

    @functools.partial(
        pl.kernel,
        out_type=jax.ShapeDtypeStruct((NC, N, D), jnp.float32),
        mesh=mesh,
        scratch_types=[pltpu.VMEM_SHARED((N, D), jnp.float32)]
        + [pltpu.VMEM((K,), jnp.int32) for _ in range(2 * NI)]
        + [pltpu.VMEM((K, D), jnp.float32) for _ in range(NB)]
        + [pltpu.SemaphoreType.DMA for _ in range(NI + 2 * NB)],
    )
    def k(data_hbm, se_hbm, de_hbm, zero_hbm, out_hbm, acc, *scr):
        srcb = scr[0:NI]
        dstb = scr[NI:2 * NI]
        rows = scr[2 * NI:2 * NI + NB]
        isem = scr[2 * NI + NB:2 * NI + NB + NI]
        gsem = scr[2 * NI + NB + NI:2 * NI + NB + NI + NB]
        ssem = scr[2 * NI + NB + NI + NB:]
        c = lax.axis_index("c")
        s = lax.axis_index("s")
        wid = s * NC + c

        pltpu.sync_copy(
            zero_hbm.at[pl.ds(s * RPT, RPT)], acc.at[pl.ds(s * RPT, RPT)]
        )

        @pl.when(s == 0)
        def _zero_rem():
            pltpu.sync_copy(
                zero_hbm.at[pl.ds(RPT * NS, REM)], acc.at[pl.ds(RPT * NS, REM)]
            )

        plsc.subcore_barrier()

        base0 = wid * EPW

        def start_idx(g, bi):
            pltpu.async_copy(se_hbm.at[pl.ds(base0 + g * K, K)], srcb[bi], isem[bi])
            pltpu.async_copy(de_hbm.at[pl.ds(base0 + g * K, K)], dstb[bi], isem[bi])

        def wait_idx(g, bi):
            pltpu.make_async_copy(
                se_hbm.at[pl.ds(base0 + g * K, K)], srcb[bi], isem[bi]
            ).wait()
            pltpu.make_async_copy(
                de_hbm.at[pl.ds(base0 + g * K, K)], dstb[bi], isem[bi]
            ).wait()

        def start_gather(b, bi):
            pltpu.async_copy(data_hbm.at[srcb[bi]], rows[b], gsem[b])

        def wait_gather(b, bi):
            pltpu.make_async_copy(data_hbm.at[srcb[bi]], rows[b], gsem[b]).wait()

        def start_scatter(b, bi):
            pltpu.async_copy(rows[b], acc.at[dstb[bi]], ssem[b], add=True)

        def wait_scatter(b, bi):
            pltpu.make_async_copy(rows[b], acc.at[dstb[bi]], ssem[b]).wait()

        for g in range(NB):
            start_idx(g, g % NI)
        for g in range(NB - 1):
            wait_idx(g, g % NI)
            start_gather(g % NB, g % NI)

        for g in range(NI):
            wait_gather(g % NB, g % NI)
            if g > 0:
                wait_scatter((g - 1) % NB, (g - 1) % NI)
            wait_idx(g + 3, (g + 3) % NI)
            start_gather((g + 3) % NB, (g + 3) % NI)
            start_scatter(g % NB, g % NI)
            start_idx(g + NB, (g + NB) % NI)

        @pl.loop(NI, MAIN, step=NI)
        def _grp(g0):
            for j in range(NI):
                g = g0 + j
                wait_gather(j % NB, j)
                wait_scatter((j - 1) % NB, (j - 1) % NI)
                wait_idx(g + 3, (j + 3) % NI)
                start_gather((j + 3) % NB, (j + 3) % NI)
                start_scatter(j % NB, j)
                start_idx(g + NB, (j + NB) % NI)

        for g in range(MAIN, NCHUNK):
            b = g % NB
            bi = g % NI
            wait_gather(b, bi)
            wait_scatter((g - 1) % NB, (g - 1) % NI)
            if g + 3 < NCHUNK:
                wait_idx(g + 3, (g + 3) % NI)
                start_gather((g + 3) % NB, (g + 3) % NI)
            start_scatter(b, bi)
            if g + NB < NCHUNK:
                start_idx(g + NB, (g + NB) % NI)
        wait_scatter((NCHUNK - 1) % NB, (NCHUNK - 1) % NI)

        plsc.subcore_barrier()
        pltpu.sync_copy(
            acc.at[pl.ds(s * RPT, RPT)], out_hbm.at[c, pl.ds(s * RPT, RPT)]
        )

        @pl.when(s == 0)
        def _flush_rem():
            pltpu.sync_copy(
                acc.at[pl.ds(RPT * NS, REM)], out_hbm.at[c, pl.ds(RPT * NS, REM)]
            )

    return k(data, se, de, zeros)


def _combine(partial):
    def body(p_ref, o_ref):
        o_ref[...] = p_ref[0] + p_ref[1]

    return pl.pallas_call(
        body,
        out_shape=jax.ShapeDtypeStruct((N, D), jnp.float32),
        grid=(5,),
        in_specs=[pl.BlockSpec((2, 2000, D), lambda i: (0, i, 0))],
        out_specs=pl.BlockSpec((2000, D), lambda i: (i, 0)),
    )(partial)


@jax.jit
def kernel(data, edge_index):
    se = edge_index[0]
    de = edge_index[1]
    zeros = jnp.zeros((N, D), jnp.float32)
    partial = _sc_partial(data, se, de, zeros)
    return _combine(partial)


# speedup vs baseline: 1.3437x; 1.0229x over previous
import functools

import jax
import jax.numpy as jnp
from jax import lax
from jax.experimental import pallas as pl
from jax.experimental.pallas import tpu as pltpu
from jax.experimental.pallas import tpu_sc as plsc

N = 10000
E = 320000
D = 128
NC = 2
NS = 16
NW = NC * NS
EPW = E // NW
K = 80
NCHUNK = EPW // K
NB = 4
NI = 8
MAIN = 120
RPT = 624
REM = N - RPT * NS


def _sc_partial(data, se, de, zeros):
    mesh = plsc.VectorSubcoreMesh(
        core_axis_name="c", subcore_axis_name="s", num_cores=NC
    )

    @functools.partial(
        pl.kernel,
        out_type=jax.ShapeDtypeStruct((NC, N, D), jnp.float32),
        mesh=mesh,
        scratch_types=[pltpu.VMEM_SHARED((N, D), jnp.float32)]
        + [pltpu.VMEM((K,), jnp.int32) for _ in range(2 * NI)]
        + [pltpu.VMEM((K, D), jnp.float32) for _ in range(NB)]
        + [pltpu.SemaphoreType.DMA for _ in range(NI + 2 * NB + 1)],
    )
    def k(data_hbm, se_hbm, de_hbm, zero_hbm, out_hbm, acc, *scr):
        srcb = scr[0:NI]
        dstb = scr[NI:2 * NI]
        rows = scr[2 * NI:2 * NI + NB]
        isem = scr[2 * NI + NB:2 * NI + NB + NI]
        gsem = scr[2 * NI + NB + NI:2 * NI + NB + NI + NB]
        ssem = scr[2 * NI + NB + NI + NB:2 * NI + NB + NI + 2 * NB]
        zsem = scr[2 * NI + NB + NI + 2 * NB]
        c = lax.axis_index("c")
        s = lax.axis_index("s")
        wid = s * NC + c

        pltpu.async_copy(
            zero_hbm.at[pl.ds(s * RPT, RPT)], acc.at[pl.ds(s * RPT, RPT)], zsem
        )

        @pl.when(s == 0)
        def _zero_rem():
            pltpu.async_copy(
                zero_hbm.at[pl.ds(RPT * NS, REM)], acc.at[pl.ds(RPT * NS, REM)],
                zsem,
            )

        base0 = wid * EPW

        def start_idx(g, bi):
            pltpu.async_copy(se_hbm.at[pl.ds(base0 + g * K, K)], srcb[bi], isem[bi])
            pltpu.async_copy(de_hbm.at[pl.ds(base0 + g * K, K)], dstb[bi], isem[bi])

        def wait_idx(g, bi):
            pltpu.make_async_copy(
                se_hbm.at[pl.ds(base0 + g * K, K)], srcb[bi], isem[bi]
            ).wait()
            pltpu.make_async_copy(
                de_hbm.at[pl.ds(base0 + g * K, K)], dstb[bi], isem[bi]
            ).wait()

        def start_gather(b, bi):
            pltpu.async_copy(data_hbm.at[srcb[bi]], rows[b], gsem[b])

        def wait_gather(b, bi):
            pltpu.make_async_copy(data_hbm.at[srcb[bi]], rows[b], gsem[b]).wait()

        def start_scatter(b, bi):
            pltpu.async_copy(rows[b], acc.at[dstb[bi]], ssem[b], add=True)

        def wait_scatter(b, bi):
            pltpu.make_async_copy(rows[b], acc.at[dstb[bi]], ssem[b]).wait()

        for g in range(NB):
            start_idx(g, g % NI)
        for g in range(NB - 1):
            wait_idx(g, g % NI)
            start_gather(g % NB, g % NI)

        pltpu.make_async_copy(
            zero_hbm.at[pl.ds(s * RPT, RPT)], acc.at[pl.ds(s * RPT, RPT)], zsem
        ).wait()

        @pl.when(s == 0)
        def _zero_rem_wait():
            pltpu.make_async_copy(
                zero_hbm.at[pl.ds(RPT * NS, REM)], acc.at[pl.ds(RPT * NS, REM)],
                zsem,
            ).wait()

        plsc.subcore_barrier()

        for g in range(NI):
            wait_gather(g % NB, g % NI)
            if g > 0:
                wait_scatter((g - 1) % NB, (g - 1) % NI)
            wait_idx(g + 3, (g + 3) % NI)
            start_gather((g + 3) % NB, (g + 3) % NI)
            start_scatter(g % NB, g % NI)
            start_idx(g + NB, (g + NB) % NI)

        @pl.loop(NI, MAIN, step=NI)
        def _grp(g0):
            for j in range(NI):
                g = g0 + j
                wait_gather(j % NB, j)
                wait_scatter((j - 1) % NB, (j - 1) % NI)
                wait_idx(g + 3, (j + 3) % NI)
                start_gather((j + 3) % NB, (j + 3) % NI)
                start_scatter(j % NB, j)
                start_idx(g + NB, (j + NB) % NI)

        for g in range(MAIN, NCHUNK):
            b = g % NB
            bi = g % NI
            wait_gather(b, bi)
            wait_scatter((g - 1) % NB, (g - 1) % NI)
            if g + 3 < NCHUNK:
                wait_idx(g + 3, (g + 3) % NI)
                start_gather((g + 3) % NB, (g + 3) % NI)
            start_scatter(b, bi)
            if g + NB < NCHUNK:
                start_idx(g + NB, (g + NB) % NI)
        wait_scatter((NCHUNK - 1) % NB, (NCHUNK - 1) % NI)

        plsc.subcore_barrier()
        pltpu.sync_copy(
            acc.at[pl.ds(s * RPT, RPT)], out_hbm.at[c, pl.ds(s * RPT, RPT)]
        )

        @pl.when(s == 0)
        def _flush_rem():
            pltpu.sync_copy(
                acc.at[pl.ds(RPT * NS, REM)], out_hbm.at[c, pl.ds(RPT * NS, REM)]
            )

    return k(data, se, de, zeros)


def _combine(partial):
    def body(p_ref, o_ref):
        o_ref[...] = p_ref[0] + p_ref[1]

    return pl.pallas_call(
        body,
        out_shape=jax.ShapeDtypeStruct((N, D), jnp.float32),
        grid=(5,),
        in_specs=[pl.BlockSpec((2, 2000, D), lambda i: (0, i, 0))],
        out_specs=pl.BlockSpec((2000, D), lambda i: (i, 0)),
    )(partial)


@jax.jit
def kernel(data, edge_index):
    se = edge_index[0]
    de = edge_index[1]
    zeros = jnp.zeros((N, D), jnp.float32)
    partial = _sc_partial(data, se, de, zeros)
    return _combine(partial)
